# baseline (speedup 1.0000x reference)
import functools

import jax
import jax.numpy as jnp
from jax import lax
from jax.experimental import pallas as pl
from jax.experimental.pallas import tpu as pltpu
from jax.experimental.pallas import tpu_sc as plsc

_NC = 2
_NS = 16
_NW = _NC * _NS

_D = 64
_S = 200
_CH = 128
_SEQ_BLK = 32


def _sc_body(n_per_w, xe, xo, tok_tbl, out, xie0, xie1, xio0, xio1,
             tbe0, tbe1, tbo0, tbo1, sg0, sg1, sw0, sw1):
    xie = (xie0, xie1)
    xio = (xio0, xio1)
    tbe = (tbe0, tbe1)
    tbo = (tbo0, tbo1)
    sg = (sg0, sg1)
    sw = (sw0, sw1)
    wid = lax.axis_index("s") * _NC + lax.axis_index("c")
    base0 = wid * n_per_w
    nblk = n_per_w

    def stage(blk, p):
        pltpu.sync_copy(xe.at[base0 + blk], xie[p])
        pltpu.sync_copy(xo.at[base0 + blk], xio[p])

    def gather_copies(p):
        return [
            pltpu.make_async_copy(tok_tbl.at[xie[p]], tbe[p], sg[p]),
            pltpu.make_async_copy(tok_tbl.at[xio[p]], tbo[p], sg[p]),
        ]

    def fire_gather(p):
        for c in gather_copies(p):
            c.start()

    def wait_gather(p):
        for c in gather_copies(p):
            c.wait()

    def wb_copies(blk, p):
        row0 = pl.multiple_of((base0 + blk) * (_S // 2), _S // 2)
        rows = pl.ds(row0, _S // 2)
        return [
            pltpu.make_async_copy(tbe[p], out.at[rows, pl.ds(0, _D)], sw[p]),
            pltpu.make_async_copy(tbo[p], out.at[rows, pl.ds(_D, _D)], sw[p]),
        ]

    def wb_start(blk, p):
        for c in wb_copies(blk, p):
            c.start()

    def wb_wait(blk, p):
        for c in wb_copies(blk, p):
            c.wait()

    stage(0, 0)
    fire_gather(0)
    stage(1, 1)
    fire_gather(1)
    wait_gather(0)
    wb_start(0, 0)

    def step(i2, carry):
        for off in (0, 1):
            b = 1 + 2 * i2 + off
            p = (1 + off) % 2
            pp = 1 - p
            stage(b + 1, pp)
            wb_wait(b - 1, pp)
            fire_gather(pp)
            wait_gather(p)
            wb_start(b, p)
        return carry

    lax.fori_loop(0, (nblk - 2) // 2, step, 0)

    wb_wait(nblk - 2, 0)
    wait_gather(1)
    wb_start(nblk - 1, 1)
    wb_wait(nblk - 1, 1)


def _tc_body(tok_ref, sege_ref, sego_ref, pose_ref, poso_ref, st_ref,
             g_ref, b_ref, out_ref):
    half = _S // 2
    xp = tok_ref[...]
    s0 = st_ref[0]
    d01 = st_ref[1] - st_ref[0]
    g = g_ref[...]
    b = b_ref[...]

    def ln(xh, pos_h, seg_ref_h):
        x3 = xh.reshape(_SEQ_BLK, half, _D)
        segf = seg_ref_h[...].astype(jnp.float32)
        h = (x3 + pos_h[...][None, :, :] + s0[None, None, :]
             + segf[:, :, None] * d01[None, None, :])
        mean = jnp.mean(h, axis=-1, keepdims=True)
        var = jnp.mean(h * h, axis=-1, keepdims=True) - mean * mean
        inv = lax.rsqrt(var + 1e-5)
        return (h - mean) * inv * g + b

    ye = ln(xp[:, :_D], pose_ref, sege_ref)
    yo = ln(xp[:, _D:], poso_ref, sego_ref)
    y = jnp.concatenate([ye[:, :, None, :], yo[:, :, None, :]], axis=2)
    out_ref[...] = y.reshape(_SEQ_BLK, _S, _D)


def kernel(x, seg, tok_table, pos_table, seg_table, gamma, beta):
    B, S = x.shape
    D = tok_table.shape[1]
    N = B * S
    n_per_w = B // _NW

    xf = x.astype(jnp.int32)
    xe = xf[:, 0::2]
    xo = xf[:, 1::2]

    mesh = plsc.VectorSubcoreMesh(core_axis_name="c", subcore_axis_name="s")
    gather = pl.kernel(
        functools.partial(_sc_body, n_per_w),
        out_type=jax.ShapeDtypeStruct((N // 2, 2 * D), jnp.float32),
        mesh=mesh,
        compiler_params=pltpu.CompilerParams(
            needs_layout_passes=False, use_tc_tiling_on_sc=False),
        scratch_types=[
            pltpu.VMEM((_S // 2,), jnp.int32),
            pltpu.VMEM((_S // 2,), jnp.int32),
            pltpu.VMEM((_S // 2,), jnp.int32),
            pltpu.VMEM((_S // 2,), jnp.int32),
            pltpu.VMEM((_S // 2, _D), jnp.float32),
            pltpu.VMEM((_S // 2, _D), jnp.float32),
            pltpu.VMEM((_S // 2, _D), jnp.float32),
            pltpu.VMEM((_S // 2, _D), jnp.float32),
            pltpu.SemaphoreType.DMA,
            pltpu.SemaphoreType.DMA,
            pltpu.SemaphoreType.DMA,
            pltpu.SemaphoreType.DMA,
        ],
    )
    packed = gather(xe, xo, tok_table)

    rows_per_blk = _SEQ_BLK * (S // 2)
    ln = pl.pallas_call(
        _tc_body,
        out_shape=jax.ShapeDtypeStruct((B, S, D), jnp.float32),
        grid=(B // _SEQ_BLK,),
        in_specs=[
            pl.BlockSpec((rows_per_blk, 2 * D), lambda i: (i, 0)),
            pl.BlockSpec((_SEQ_BLK, S // 2), lambda i: (i, 0)),
            pl.BlockSpec((_SEQ_BLK, S // 2), lambda i: (i, 0)),
            pl.BlockSpec((S // 2, D), lambda i: (0, 0)),
            pl.BlockSpec((S // 2, D), lambda i: (0, 0)),
            pl.BlockSpec((2, D), lambda i: (0, 0)),
            pl.BlockSpec((D,), lambda i: (0,)),
            pl.BlockSpec((D,), lambda i: (0,)),
        ],
        out_specs=pl.BlockSpec((_SEQ_BLK, S, D), lambda i: (i, 0, 0)),
    )
    segi = seg.astype(jnp.int32)
    return ln(packed, segi[:, 0::2], segi[:, 1::2],
              pos_table[0:S:2], pos_table[1:S:2], seg_table, gamma, beta)


# speedup vs baseline: 1.7658x; 1.7658x over previous
import functools

import jax
import jax.numpy as jnp
from jax import lax
from jax.experimental import pallas as pl
from jax.experimental.pallas import tpu as pltpu
from jax.experimental.pallas import tpu_sc as plsc

_NC = 2
_NS = 16
_NW = _NC * _NS

_D = 64
_S = 200
_CH = 128
_SEQ_BLK = 32


def _sc_body(n_per_w, xf, tok_tbl, out, xi0, xi1, tb0, tb1, sg0, sg1,
             sw0, sw1):
    xi = (xi0, xi1)
    tb = (tb0, tb1)
    sg = (sg0, sg1)
    sw = (sw0, sw1)
    wid = lax.axis_index("s") * _NC + lax.axis_index("c")
    base0 = wid * n_per_w
    nblk = n_per_w

    def stage(blk, p):
        pltpu.sync_copy(xf.at[base0 + blk], xi[p])

    def gather_copies(p):
        cps = []
        for (o, n) in ((0, _CH), (_CH, _S - _CH)):
            sl = pl.ds(o, n)
            cps.append(pltpu.make_async_copy(
                tok_tbl.at[xi[p].at[sl]], tb[p].at[sl], sg[p]))
        return cps

    def fire_gather(p):
        for c in gather_copies(p):
            c.start()

    def wait_gather(p):
        for c in gather_copies(p):
            c.wait()

    def wb_copy(blk, p):
        row0 = pl.multiple_of((base0 + blk) * _S, _S)
        return pltpu.make_async_copy(
            tb[p], out.at[pl.ds(row0, _S), pl.ds(0, _D)], sw[p])

    stage(0, 0)
    fire_gather(0)
    stage(1, 1)
    fire_gather(1)
    wait_gather(0)
    wb_copy(0, 0).start()

    def step(i2, carry):
        for off in (0, 1):
            b = 1 + 2 * i2 + off
            p = (1 + off) % 2
            pp = 1 - p
            stage(b + 1, pp)
            wb_copy(b - 1, pp).wait()
            fire_gather(pp)
            wait_gather(p)
            wb_copy(b, p).start()
        return carry

    lax.fori_loop(0, (nblk - 2) // 2, step, 0)

    wb_copy(nblk - 2, 0).wait()
    wait_gather(1)
    wb_copy(nblk - 1, 1).start()
    wb_copy(nblk - 1, 1).wait()


def _tc_body(tok_ref, seg_ref, pos_ref, st_ref, g_ref, b_ref, out_ref):
    x = tok_ref[...][:, :_D].reshape(_SEQ_BLK, _S, _D)
    segf = seg_ref[...].astype(jnp.float32)
    s0 = st_ref[0]
    d01 = st_ref[1] - st_ref[0]
    h = (x + pos_ref[...][None, :, :] + s0[None, None, :]
         + segf[:, :, None] * d01[None, None, :])
    mean = jnp.mean(h, axis=-1, keepdims=True)
    var = jnp.mean(h * h, axis=-1, keepdims=True) - mean * mean
    inv = lax.rsqrt(var + 1e-5)
    out_ref[...] = (h - mean) * inv * g_ref[...] + b_ref[...]


def kernel(x, seg, tok_table, pos_table, seg_table, gamma, beta):
    B, S = x.shape
    D = tok_table.shape[1]
    N = B * S
    n_per_w = B // _NW

    xf = x.astype(jnp.int32)

    mesh = plsc.VectorSubcoreMesh(core_axis_name="c", subcore_axis_name="s")
    gather = pl.kernel(
        functools.partial(_sc_body, n_per_w),
        out_type=jax.ShapeDtypeStruct((N, 2 * D), jnp.float32),
        mesh=mesh,
        compiler_params=pltpu.CompilerParams(
            needs_layout_passes=False, use_tc_tiling_on_sc=False),
        scratch_types=[
            pltpu.VMEM((_S,), jnp.int32),
            pltpu.VMEM((_S,), jnp.int32),
            pltpu.VMEM((_S, _D), jnp.float32),
            pltpu.VMEM((_S, _D), jnp.float32),
            pltpu.SemaphoreType.DMA,
            pltpu.SemaphoreType.DMA,
            pltpu.SemaphoreType.DMA,
            pltpu.SemaphoreType.DMA,
        ],
    )
    padded = gather(xf, tok_table)

    ln = pl.pallas_call(
        _tc_body,
        out_shape=jax.ShapeDtypeStruct((B, S, D), jnp.float32),
        grid=(B // _SEQ_BLK,),
        in_specs=[
            pl.BlockSpec((_SEQ_BLK * S, 2 * D), lambda i: (i, 0)),
            pl.BlockSpec((_SEQ_BLK, S), lambda i: (i, 0)),
            pl.BlockSpec((S, D), lambda i: (0, 0)),
            pl.BlockSpec((2, D), lambda i: (0, 0)),
            pl.BlockSpec((D,), lambda i: (0,)),
            pl.BlockSpec((D,), lambda i: (0,)),
        ],
        out_specs=pl.BlockSpec((_SEQ_BLK, S, D), lambda i: (i, 0, 0)),
    )
    return ln(padded, seg.astype(jnp.int32), pos_table[:S], seg_table,
              gamma, beta)


# speedup vs baseline: 1.9919x; 1.1280x over previous
import functools

import jax
import jax.numpy as jnp
from jax import lax
from jax.experimental import pallas as pl
from jax.experimental.pallas import tpu as pltpu
from jax.experimental.pallas import tpu_sc as plsc

_NC = 2
_NS = 16
_NW = _NC * _NS

_D = 64
_S = 200
_CH = 128
_SEQ_BLK = 32


def _sc_body(n_per_w, xf, tok_tbl, out, xi0, xi1, tb0, tb1, sg0, sg1,
             sw0, sw1):
    xi = (xi0, xi1)
    tb = (tb0, tb1)
    sg = (sg0, sg1)
    sw = (sw0, sw1)
    wid = lax.axis_index("s") * _NC + lax.axis_index("c")
    base0 = wid * n_per_w
    nblk = n_per_w

    def stage(blk, p):
        pltpu.sync_copy(xf.at[base0 + blk], xi[p])

    def gather_copies(p):
        cps = []
        for (o, n) in ((0, _CH), (_CH, _S - _CH)):
            sl = pl.ds(o, n)
            cps.append(pltpu.make_async_copy(
                tok_tbl.at[xi[p].at[sl]], tb[p].at[sl], sg[p]))
        return cps

    def fire_gather(p):
        for c in gather_copies(p):
            c.start()

    def wait_gather(p):
        for c in gather_copies(p):
            c.wait()

    def wb_copy(blk, p):
        row0 = pl.multiple_of((base0 + blk) * _S, _S)
        return pltpu.make_async_copy(
            tb[p], out.at[pl.ds(row0, _S), pl.ds(0, _D)], sw[p])

    stage(0, 0)
    fire_gather(0)
    stage(1, 1)
    fire_gather(1)
    wait_gather(0)
    wb_copy(0, 0).start()

    def step(i2, carry):
        for off in (0, 1):
            b = 1 + 2 * i2 + off
            p = (1 + off) % 2
            pp = 1 - p
            stage(b + 1, pp)
            wb_copy(b - 1, pp).wait()
            fire_gather(pp)
            wait_gather(p)
            wb_copy(b, p).start()
        return carry

    lax.fori_loop(0, (nblk - 2) // 2, step, 0)

    wb_copy(nblk - 2, 0).wait()
    wait_gather(1)
    wb_copy(nblk - 1, 1).start()
    wb_copy(nblk - 1, 1).wait()


def _tc_body(tok_ref, seg_ref, pos_ref, st_ref, g_ref, b_ref, out_ref):
    x = tok_ref[...][:, :_D].reshape(_SEQ_BLK, _S, _D)
    segf = seg_ref[...].astype(jnp.float32)
    s0 = st_ref[0]
    d01 = st_ref[1] - st_ref[0]
    h = (x + pos_ref[...][None, :, :] + s0[None, None, :]
         + segf[:, :, None] * d01[None, None, :])
    mean = jnp.mean(h, axis=-1, keepdims=True)
    var = jnp.mean(h * h, axis=-1, keepdims=True) - mean * mean
    inv = lax.rsqrt(var + 1e-5)
    y = (h - mean) * inv * g_ref[...] + b_ref[...]
    out_ref[...] = y.reshape(_SEQ_BLK * _S, _D)


def kernel(x, seg, tok_table, pos_table, seg_table, gamma, beta):
    B, S = x.shape
    D = tok_table.shape[1]
    N = B * S
    n_per_w = B // _NW

    xf = x.astype(jnp.int32)

    mesh = plsc.VectorSubcoreMesh(core_axis_name="c", subcore_axis_name="s")
    gather = pl.kernel(
        functools.partial(_sc_body, n_per_w),
        out_type=jax.ShapeDtypeStruct((N, 2 * D), jnp.float32),
        mesh=mesh,
        compiler_params=pltpu.CompilerParams(
            needs_layout_passes=False, use_tc_tiling_on_sc=False),
        scratch_types=[
            pltpu.VMEM((_S,), jnp.int32),
            pltpu.VMEM((_S,), jnp.int32),
            pltpu.VMEM((_S, _D), jnp.float32),
            pltpu.VMEM((_S, _D), jnp.float32),
            pltpu.SemaphoreType.DMA,
            pltpu.SemaphoreType.DMA,
            pltpu.SemaphoreType.DMA,
            pltpu.SemaphoreType.DMA,
        ],
    )
    padded = gather(xf, tok_table)

    ln = pl.pallas_call(
        _tc_body,
        out_shape=jax.ShapeDtypeStruct((N, D), jnp.float32),
        grid=(B // _SEQ_BLK,),
        in_specs=[
            pl.BlockSpec((_SEQ_BLK * S, 2 * D), lambda i: (i, 0)),
            pl.BlockSpec((_SEQ_BLK, S), lambda i: (i, 0)),
            pl.BlockSpec((S, D), lambda i: (0, 0)),
            pl.BlockSpec((2, D), lambda i: (0, 0)),
            pl.BlockSpec((D,), lambda i: (0,)),
            pl.BlockSpec((D,), lambda i: (0,)),
        ],
        out_specs=pl.BlockSpec((_SEQ_BLK * S, D), lambda i: (i, 0)),
    )
    out = ln(padded, seg.astype(jnp.int32), pos_table[:S], seg_table,
             gamma, beta)
    return out.reshape(B, S, D)
